# Initial kernel scaffold; baseline (speedup 1.0000x reference)
import jax
import jax.numpy as jnp
from jax.experimental import pallas as pl


def kernel(seq_len, bias_table):
    raise NotImplementedError("write your pallas kernel here")


# speedup vs baseline: 42.7711x; 42.7711x over previous
import functools

import jax
import jax.numpy as jnp
from jax import lax
from jax.experimental import pallas as pl
from jax.experimental.pallas import tpu as pltpu
from jax.experimental.pallas import tpu_sc as plsc

N_HEADS = 16
SEQ = 2048
TAB = 255
GLEN = 2 * SEQ - 1
SLOT = 4352
NC = 2
NS = 16
N_TILES = NC * NS
ROWS_PER_TILE = N_HEADS * SEQ // N_TILES
FIRE = 16


def _sc_body(table_hbm, out_hbm, table_v, g8_v, sem):
    cid = lax.axis_index("c")
    sid = lax.axis_index("s")
    wid = sid * NC + cid
    head = wid >> 1
    half = wid & 1

    pltpu.sync_copy(table_hbm, table_v)

    lane = lax.iota(jnp.int32, 16)

    for s in range(8):
        @pl.loop(0, SLOT // 16)
        def _build(c, s=s):
            p = c * 16 + lane
            idx = jnp.clip(p + (s - (SEQ - 128)), 0, TAB - 1) * N_HEADS + head
            vals = plsc.load_gather(table_v, [idx])
            g8_v[pl.ds(s * SLOT + c * 16, 16)] = vals

    row0 = head * SEQ + half * (SEQ // 2)

    @pl.loop(0, ROWS_PER_TILE // FIRE)
    def _rows(r):
        copies = []
        for k in range(FIRE):
            i_local = r * FIRE + k
            i = half * (SEQ // 2) + i_local
            start = (SEQ - 1) - i
            s = (7 - k) & 7
            base = pl.multiple_of(start - s, 8)
            src = g8_v.at[pl.ds(s * SLOT + base, SEQ)]
            dst = out_hbm.at[pl.ds((row0 + i_local) * SEQ, SEQ)]
            copies.append(pltpu.async_copy(src, dst, sem))
        for c in copies:
            c.wait()


@jax.jit
def _relative_pos_bias_sc(bias_table):
    mesh = plsc.VectorSubcoreMesh(core_axis_name="c", subcore_axis_name="s")
    fn = pl.kernel(
        _sc_body,
        out_type=jax.ShapeDtypeStruct((N_HEADS * SEQ * SEQ,), jnp.float32),
        mesh=mesh,
        scratch_types=[
            pltpu.VMEM((TAB * N_HEADS,), jnp.float32),
            pltpu.VMEM((8 * SLOT,), jnp.float32),
            pltpu.SemaphoreType.DMA,
        ],
        compiler_params=pltpu.CompilerParams(needs_layout_passes=False),
    )
    out_flat = fn(bias_table.reshape(-1))
    return out_flat.reshape(1, N_HEADS, SEQ, SEQ)


def kernel(seq_len, bias_table):
    del seq_len
    return _relative_pos_bias_sc(bias_table)
